# baseline (speedup 1.0000x reference)
import functools

import jax
import jax.numpy as jnp
from jax import lax
from jax.experimental import pallas as pl
from jax.experimental.pallas import tpu as pltpu
from jax.experimental.pallas import tpu_sc as plsc

_B = 16384
_D = 32
_NC = 2
_NS = 16
_NW = _NC * _NS
_BPW = _B // _NW
_L = 16
_CH = 128
_NCH = _BPW // _CH


def _mf_body(users_hbm, items_hbm, uemb_hbm, iemb_hbm, ubias_hbm,
             ibias_hbm, out_hbm, uidx_v, iidx_v, urows_v, irows_v, ub_v,
             ib_v, out_v, sem):
    wid = lax.axis_index("s") * _NC + lax.axis_index("c")
    base = wid * _BPW

    pltpu.sync_copy(users_hbm.at[pl.ds(base, _BPW)], uidx_v)
    pltpu.sync_copy(items_hbm.at[pl.ds(base, _BPW)], iidx_v)

    cps = []
    for g in range(_NCH):
        uix = uidx_v.at[pl.ds(g * _CH, _CH)]
        iix = iidx_v.at[pl.ds(g * _CH, _CH)]
        cps.append(pltpu.async_copy(
            uemb_hbm.at[uix], urows_v.at[pl.ds(g * _CH, _CH)], sem))
        cps.append(pltpu.async_copy(
            iemb_hbm.at[iix], irows_v.at[pl.ds(g * _CH, _CH)], sem))
        cps.append(pltpu.async_copy(
            ubias_hbm.at[uix], ub_v.at[pl.ds(g * _CH, _CH)], sem))
        cps.append(pltpu.async_copy(
            ibias_hbm.at[iix], ib_v.at[pl.ds(g * _CH, _CH)], sem))
    for cp in cps:
        cp.wait()

    def subgroup(s, carry):
        b0 = s * _L
        rvec = b0 + lax.iota(jnp.int32, _L)
        acc = ub_v[pl.ds(b0, _L)] + ib_v[pl.ds(b0, _L)]
        for d in range(_D):
            dvec = jnp.full((_L,), d, jnp.int32)
            uv = plsc.load_gather(urows_v, [rvec, dvec])
            iv = plsc.load_gather(irows_v, [rvec, dvec])
            acc = acc + uv * iv
        out_v[pl.ds(b0, _L)] = acc
        return carry

    lax.fori_loop(0, _BPW // _L, subgroup, 0)

    pltpu.sync_copy(out_v, out_hbm.at[pl.ds(base, _BPW)])


@jax.jit
def _mf_call(users, items, uemb, iemb, ubias, ibias):
    mesh = plsc.VectorSubcoreMesh(core_axis_name="c", subcore_axis_name="s")
    k = pl.kernel(
        _mf_body,
        out_type=jax.ShapeDtypeStruct((_B,), jnp.float32),
        mesh=mesh,
        compiler_params=pltpu.CompilerParams(
            needs_layout_passes=False, use_tc_tiling_on_sc=False),
        scratch_types=[
            pltpu.VMEM((_BPW,), jnp.int32),
            pltpu.VMEM((_BPW,), jnp.int32),
            pltpu.VMEM((_BPW, _D), jnp.float32),
            pltpu.VMEM((_BPW, _D), jnp.float32),
            pltpu.VMEM((_BPW,), jnp.float32),
            pltpu.VMEM((_BPW,), jnp.float32),
            pltpu.VMEM((_BPW,), jnp.float32),
            pltpu.SemaphoreType.DMA,
        ],
    )
    return k(users, items, uemb, iemb, ubias, ibias)


def kernel(users, items, user_emb, item_emb, user_bias_tab, item_bias_tab):
    users = users.astype(jnp.int32)
    items = items.astype(jnp.int32)
    ubias = user_bias_tab.reshape(-1)
    ibias = item_bias_tab.reshape(-1)
    out = _mf_call(users, items, user_emb, item_emb, ubias, ibias)
    return out.reshape(_B, 1)


# speedup vs baseline: 1.0035x; 1.0035x over previous
import jax
import jax.numpy as jnp
from jax import lax
from jax.experimental import pallas as pl
from jax.experimental.pallas import tpu as pltpu
from jax.experimental.pallas import tpu_sc as plsc

_B = 16384
_D = 32
_NC = 2
_NS = 16
_NW = _NC * _NS
_BPW = _B // _NW
_L = 16
_CH = 128
_NCH = _BPW // _CH


def _mf_body(users_hbm, items_hbm, uemb_hbm, iemb_hbm, ubias_hbm,
             ibias_hbm, out_hbm, uidx_v, iidx_v, urows_v, irows_v, ub_v,
             ib_v, out_v, sem):
    wid = lax.axis_index("s") * _NC + lax.axis_index("c")
    base = wid * _BPW

    pltpu.sync_copy(users_hbm.at[pl.ds(base, _BPW)], uidx_v)
    pltpu.sync_copy(items_hbm.at[pl.ds(base, _BPW)], iidx_v)

    cps = []
    for g in range(_NCH):
        uix = uidx_v.at[pl.ds(g * _CH, _CH)]
        iix = iidx_v.at[pl.ds(g * _CH, _CH)]
        cps.append(pltpu.async_copy(
            uemb_hbm.at[uix], urows_v.at[pl.ds(g * _CH, _CH)], sem))
        cps.append(pltpu.async_copy(
            iemb_hbm.at[iix], irows_v.at[pl.ds(g * _CH, _CH)], sem))
        cps.append(pltpu.async_copy(
            ubias_hbm.at[uix], ub_v.at[pl.ds(g * _CH, _CH)], sem))
        cps.append(pltpu.async_copy(
            ibias_hbm.at[iix], ib_v.at[pl.ds(g * _CH, _CH)], sem))
    for cp in cps:
        cp.wait()

    def subgroup(s, carry):
        b0 = s * _L
        rvec = b0 + lax.iota(jnp.int32, _L)
        acc = ub_v[pl.ds(b0, _L)] + ib_v[pl.ds(b0, _L)]
        for d in range(_D):
            dvec = jnp.full((_L,), d, jnp.int32)
            uv = plsc.load_gather(urows_v, [rvec, dvec])
            iv = plsc.load_gather(irows_v, [rvec, dvec])
            acc = acc + uv * iv
        out_v[pl.ds(b0, _L)] = acc
        return carry

    lax.fori_loop(0, _BPW // _L, subgroup, 0)

    pltpu.sync_copy(out_v, out_hbm.at[pl.ds(base, _BPW)])


@jax.jit
def _mf_call(users, items, uemb, iemb, ubias, ibias):
    mesh = plsc.VectorSubcoreMesh(core_axis_name="c", subcore_axis_name="s")
    k = pl.kernel(
        _mf_body,
        out_type=jax.ShapeDtypeStruct((_B,), jnp.float32),
        mesh=mesh,
        compiler_params=pltpu.CompilerParams(
            needs_layout_passes=False, use_tc_tiling_on_sc=False),
        scratch_types=[
            pltpu.VMEM((_BPW,), jnp.int32),
            pltpu.VMEM((_BPW,), jnp.int32),
            pltpu.VMEM((_BPW, _D), jnp.float32),
            pltpu.VMEM((_BPW, _D), jnp.float32),
            pltpu.VMEM((_BPW,), jnp.float32),
            pltpu.VMEM((_BPW,), jnp.float32),
            pltpu.VMEM((_BPW,), jnp.float32),
            pltpu.SemaphoreType.DMA,
        ],
    )
    return k(users, items, uemb, iemb, ubias, ibias)


def kernel(users, items, user_emb, item_emb, user_bias_tab, item_bias_tab):
    users = users.astype(jnp.int32)
    items = items.astype(jnp.int32)
    ubias = user_bias_tab.reshape(-1)
    ibias = item_bias_tab.reshape(-1)
    out = _mf_call(users, items, user_emb, item_emb, ubias, ibias)
    return out.reshape(_B, 1)
